# baseline (speedup 1.0000x reference)
import jax
import jax.numpy as jnp
from jax import lax
from jax.experimental import pallas as pl
from jax.experimental.pallas import tpu as pltpu

B = 256
P = 1024
F = 64
D = 64
C = 16
N = 64
K = F * D
GB = 16


def _ka_body(filt_ref, x_ref, o_ref):
    xb = x_ref[...]
    fv = filt_ref[0]
    pio = lax.broadcasted_iota(jnp.int32, (P, F), 0)
    sel = (pio == fv).astype(jnp.float32)
    r = jnp.dot(xb, sel, preferred_element_type=jnp.float32,
                precision=lax.Precision.HIGHEST)
    o_ref[...] = r.reshape(GB, D, F)


def _ka(filt3, xbytes):
    return pl.pallas_call(
        _ka_body,
        grid=(B // GB,),
        in_specs=[
            pl.BlockSpec((1, 1, F), lambda g: (0, 0, 0)),
            pl.BlockSpec((GB * D, P), lambda g: (g, 0)),
        ],
        out_specs=pl.BlockSpec((GB, D, F), lambda g: (g, 0, 0)),
        out_shape=jax.ShapeDtypeStruct((B, D, F), jnp.float32),
    )(filt3, xbytes)


def _kb_body(wt_ref, xgt_ref, g_ref, o_ref, y_ref, s_ref):
    c = pl.program_id(0)
    xgt = xgt_ref[...]

    @pl.when(c == 0)
    def _():
        s_ref[0] = jnp.sum(xgt, axis=0)

    y_ref[c] = jnp.dot(wt_ref[0], xgt, preferred_element_type=jnp.float32,
                       precision=lax.Precision.HIGHEST)

    @pl.when(c == C - 1)
    def _():
        y = y_ref[...]
        s = s_ref[...]
        sinv = jnp.where(s > 0, 1.0 / s, 0.0)
        fam = jnp.max(y, axis=1)
        avg = jnp.mean(fam * sinv)
        temp = 1.0 / (avg + 0.0001) - 1.0
        scale = (sinv / temp).reshape(1, 1, B)
        z = y * scale + g_ref[...]
        m = jnp.max(z, axis=1, keepdims=True)
        iota = lax.broadcasted_iota(jnp.int32, (C, N, B), 1)
        kidx = jnp.min(jnp.where(z == m, iota, N), axis=1, keepdims=True)
        o_ref[...] = (iota == kidx).astype(jnp.float32)


def _kb(wt, xgt, gum):
    return pl.pallas_call(
        _kb_body,
        grid=(C,),
        in_specs=[
            pl.BlockSpec((1, N, K), lambda c: (c, 0, 0)),
            pl.BlockSpec((K, B), lambda c: (0, 0)),
            pl.BlockSpec((C, N, B), lambda c: (0, 0, 0)),
        ],
        out_specs=pl.BlockSpec((C, N, B), lambda c: (0, 0, 0)),
        out_shape=jax.ShapeDtypeStruct((C, N, B), jnp.float32),
        scratch_shapes=[
            pltpu.VMEM((C, N, B), jnp.float32),
            pltpu.VMEM((1, B), jnp.float32),
        ],
    )(wt, xgt, gum)


def kernel(x, weights, input_filter):
    x2d = jnp.transpose(x, (0, 2, 3, 1)).reshape(B * D, P)
    wt = jnp.transpose(weights, (0, 2, 1))
    filt3 = input_filter.astype(jnp.int32).reshape(1, 1, F)

    xsel = _ka(filt3, x2d)
    xgt = jnp.transpose(xsel, (2, 1, 0)).reshape(K, B)

    gum = jnp.transpose(
        jax.random.gumbel(jax.random.key(42), (B, C, N), jnp.float32),
        (1, 2, 0))
    oh = _kb(wt, xgt, gum)
    return jnp.transpose(oh, (2, 0, 1))


# speedup vs baseline: 1.2594x; 1.2594x over previous
import jax
import jax.numpy as jnp
from jax import lax
from jax.experimental import pallas as pl
from jax.experimental.pallas import tpu as pltpu

B = 256
P = 1024
F = 64
D = 64
C = 16
N = 64
K = F * D
GB = 32


def _ka_body(filt_ref, x_ref, o_ref):
    xb = x_ref[...]
    fv = filt_ref[0]
    pio = lax.broadcasted_iota(jnp.int32, (P, F), 0)
    sel = (pio == fv).astype(jnp.float32)
    r = jnp.dot(xb, sel, preferred_element_type=jnp.float32,
                precision=lax.Precision.HIGHEST)
    o_ref[...] = r.reshape(GB, D, F)


def _ka(filt3, xbytes):
    return pl.pallas_call(
        _ka_body,
        grid=(B // GB,),
        in_specs=[
            pl.BlockSpec((1, 1, F), lambda g: (0, 0, 0)),
            pl.BlockSpec((GB * D, P), lambda g: (g, 0)),
        ],
        out_specs=pl.BlockSpec((GB, D, F), lambda g: (g, 0, 0)),
        out_shape=jax.ShapeDtypeStruct((B, D, F), jnp.float32),
    )(filt3, xbytes)


def _kb_body(wt_ref, xgt_ref, g_ref, o_ref, xh_ref, xl_ref, y_ref, s_ref):
    c = pl.program_id(0)

    @pl.when(c == 0)
    def _():
        xg = xgt_ref[...]
        hi = xg.astype(jnp.bfloat16)
        xh_ref[...] = hi
        xl_ref[...] = (xg - hi.astype(jnp.float32)).astype(jnp.bfloat16)
        s_ref[0] = jnp.sum(xg, axis=0)

    wh = wt_ref[0].astype(jnp.bfloat16)
    y_ref[c] = (
        jnp.dot(wh, xh_ref[...], preferred_element_type=jnp.float32)
        + jnp.dot(wh, xl_ref[...], preferred_element_type=jnp.float32))

    @pl.when(c == C - 1)
    def _():
        y = y_ref[...]
        s = s_ref[...]
        sinv = jnp.where(s > 0, 1.0 / s, 0.0)
        fam = jnp.max(y, axis=1)
        avg = jnp.mean(fam * sinv)
        temp = 1.0 / (avg + 0.0001) - 1.0
        scale = (sinv / temp).reshape(1, 1, B)
        z = y * scale + g_ref[...]
        m = jnp.max(z, axis=1, keepdims=True)
        iota = lax.broadcasted_iota(jnp.int32, (C, N, B), 1)
        kidx = jnp.min(jnp.where(z == m, iota, N), axis=1, keepdims=True)
        o_ref[...] = (iota == kidx).astype(jnp.float32)


def _kb(wt, xgt, gum):
    return pl.pallas_call(
        _kb_body,
        grid=(C,),
        in_specs=[
            pl.BlockSpec((1, N, K), lambda c: (c, 0, 0)),
            pl.BlockSpec((K, B), lambda c: (0, 0)),
            pl.BlockSpec((C, N, B), lambda c: (0, 0, 0)),
        ],
        out_specs=pl.BlockSpec((C, N, B), lambda c: (0, 0, 0)),
        out_shape=jax.ShapeDtypeStruct((C, N, B), jnp.float32),
        scratch_shapes=[
            pltpu.VMEM((K, B), jnp.bfloat16),
            pltpu.VMEM((K, B), jnp.bfloat16),
            pltpu.VMEM((C, N, B), jnp.float32),
            pltpu.VMEM((1, B), jnp.float32),
        ],
    )(wt, xgt, gum)


def kernel(x, weights, input_filter):
    x2d = jnp.transpose(x, (0, 2, 3, 1)).reshape(B * D, P)
    wt = jnp.transpose(weights, (0, 2, 1))
    filt3 = input_filter.astype(jnp.int32).reshape(1, 1, F)

    xsel = _ka(filt3, x2d)
    xgt = jnp.transpose(xsel, (2, 1, 0)).reshape(K, B)

    gum = jnp.transpose(
        jax.random.gumbel(jax.random.key(42), (B, C, N), jnp.float32),
        (1, 2, 0))
    oh = _kb(wt, xgt, gum)
    return jnp.transpose(oh, (2, 0, 1))
